# baseline (speedup 1.0000x reference)
import functools

import jax
import jax.numpy as jnp
from jax import lax
from jax.experimental import pallas as pl
from jax.experimental.pallas import tpu as pltpu
from jax.experimental.pallas import tpu_sc as plsc

B = 16384
NNUM = 13
NF = 26
V = 100000
ED = 16
H = 128
EPS = 1e-5

NC, NS = 2, 16
NW = NC * NS

RPF = 12504
VPAD = RPF * 8
NCH_T = 25
CH_T = 4096
CH_LAST = 1664
VTAIL = 99968
UNITS_T = NF * NCH_T
HALF_E = 2048

ROWS = B * NF
ROWS_PER_W = ROWS // NW
CHUNK = 1664
NCHUNK = ROWS_PER_W // CHUNK


QE = 1024
QW = QE * ED
FLUSH_B = QW * 4


@functools.cache
def _sc_transpose_fn():
    mesh = plsc.VectorSubcoreMesh(core_axis_name="c", subcore_axis_name="s")

    @functools.partial(
        pl.kernel,
        out_type=jax.ShapeDtypeStruct((NF, RPF, 128), jnp.float32),
        mesh=mesh,
        scratch_types=[
            pltpu.VMEM((2, 8, CH_T), jnp.float32),
            pltpu.VMEM((2, 8, 128), jnp.float32),
            pltpu.VMEM((QE // 8, 128), jnp.float32),
            pltpu.VMEM((QE // 8, 128), jnp.float32),
            pltpu.SemaphoreType.DMA,
            pltpu.SemaphoreType.DMA,
            pltpu.SemaphoreType.DMA,
        ],
        compiler_params=pltpu.CompilerParams(needs_layout_passes=False),
    )
    def _sc_transpose(tab_hbm, tail_hbm, out_hbm, slab, tslab, obufA, obufB,
                      bsem, fsem0, fsem1):
        wid = lax.axis_index("s") * NC + lax.axis_index("c")
        u_lo = (UNITS_T * wid) // NW
        u_hi = (UNITS_T * (wid + 1)) // NW
        iot = lax.iota(jnp.int32, 16)
        shalf = [slab.at[0], slab.at[1]]
        thalf = [tslab.at[0], tslab.at[1]]
        rowc = [jnp.full((16,), e % 8, jnp.int32) for e in range(16)]
        fsems = [fsem0, fsem1]
        obufs = [obufA, obufB]

        def blocks(ob, rows, col_base, li_base, n_ent):
            def bbody(i, carry):
                ks = [i * 32 + j * 16 + iot for j in range(2)]
                vcols = [col_base + k for k in ks]
                rowvs = [(li_base + k) >> 3 for k in ks]
                colbs = [((li_base + k) & 7) << 4 for k in ks]
                for e in range(16):
                    for j in range(2):
                        vals = plsc.load_gather(rows[e // 8],
                                                [rowc[e], vcols[j]])
                        plsc.store_scatter(ob, [rowvs[j], colbs[j] + e], vals)
                return carry

            lax.fori_loop(0, n_ent // 32, bbody, 0)

        def flush_wait(h):
            pltpu.make_async_copy(obufs[h], out_hbm.at[0, pl.ds(0, QE // 8)],
                                  fsems[h]).wait()

        def do_quarter(f, c0, h0, h, n_h):
            @pl.when(n_h >= 1)
            def _():
                flush_wait(h)
            blocks(obufs[h], shalf, h0, 0, QE)
            r0 = pl.multiple_of((c0 + h0) // 8, 8)
            pltpu.async_copy(obufs[h], out_hbm.at[f, pl.ds(r0, QE // 8)],
                             fsems[h])
            return n_h + 1

        def do_chunk(u, f, c0, is_tail, n0, n1):
            c0 = pl.multiple_of(c0, 1024)
            cw = CH_LAST if is_tail else CH_T
            pltpu.async_copy(tab_hbm.at[2 * f, :, pl.ds(c0, cw)],
                             slab.at[0, :, pl.ds(0, cw)], bsem)
            pltpu.async_copy(tab_hbm.at[2 * f + 1, :, pl.ds(c0, cw)],
                             slab.at[1, :, pl.ds(0, cw)], bsem)
            pltpu.make_async_copy(tab_hbm.at[2 * f, :, pl.ds(c0, cw)],
                                  slab.at[0, :, pl.ds(0, cw)], bsem).wait()
            pltpu.make_async_copy(tab_hbm.at[2 * f + 1, :, pl.ds(c0, cw)],
                                  slab.at[1, :, pl.ds(0, cw)], bsem).wait()
            if not is_tail:
                n0 = do_quarter(f, c0, 0 * QE, 0, n0)
                n1 = do_quarter(f, c0, 1 * QE, 1, n1)
                n0 = do_quarter(f, c0, 2 * QE, 0, n0)
                n1 = do_quarter(f, c0, 3 * QE, 1, n1)
            else:
                @pl.when(n0 >= 1)
                def _():
                    flush_wait(0)
                blocks(obufA, shalf, 0, 0, QE)
                r0 = pl.multiple_of(c0 // 8, 8)
                pltpu.sync_copy(obufA, out_hbm.at[f, pl.ds(r0, QE // 8)])
                blocks(obufA, shalf, QE, 0, CH_LAST - QE)
                pltpu.sync_copy(tail_hbm.at[2 * f], tslab.at[0])
                pltpu.sync_copy(tail_hbm.at[2 * f + 1], tslab.at[1])
                blocks(obufA, thalf, 0, CH_LAST - QE, 32)
                r2 = pl.multiple_of((c0 + QE) // 8, 8)
                pltpu.sync_copy(obufA.at[pl.ds(0, 88)],
                                out_hbm.at[f, pl.ds(r2, 88)])
                n0 = n0 * 0
            return n0, n1

        def body(u, carry):
            n0, n1 = carry
            f = u // NCH_T
            c = u % NCH_T
            is_tail = c == NCH_T - 1
            c0 = c * CH_T

            def tail_fn():
                return do_chunk(u, f, (NCH_T - 1) * CH_T, True, n0, n1)

            def reg_fn():
                return do_chunk(u, f, c0, False, n0, n1)

            return lax.cond(is_tail, tail_fn, reg_fn)

        n0, n1 = lax.fori_loop(u_lo, u_hi, body, (jnp.int32(0), jnp.int32(0)))

        @pl.when(n0 >= 1)
        def _():
            flush_wait(0)

        @pl.when(n1 >= 1)
        def _():
            flush_wait(1)

    return _sc_transpose


@functools.cache
def _sc_gather_fn():
    mesh = plsc.VectorSubcoreMesh(core_axis_name="c", subcore_axis_name="s")

    @functools.partial(
        pl.kernel,
        out_type=jax.ShapeDtypeStruct((ROWS, ED), jnp.float32),
        mesh=mesh,
        scratch_types=[
            pltpu.VMEM((ROWS_PER_W,), jnp.int32),
            pltpu.VMEM((2, CHUNK, ED), jnp.float32),
            pltpu.SemaphoreType.DMA,
            pltpu.SemaphoreType.DMA,
            pltpu.SemaphoreType.DMA,
        ],
        compiler_params=pltpu.CompilerParams(use_tc_tiling_on_sc=False),
    )
    def _sc_gather(tab_hbm, idx_hbm, out_hbm, idx_v, rows_v, gsem, ssem,
                   isem):
        wid = lax.axis_index("s") * NC + lax.axis_index("c")
        base = wid * ROWS_PER_W
        pltpu.async_copy(idx_hbm.at[pl.ds(base, ROWS_PER_W)], idx_v,
                         isem).wait()

        def gather(i, buf):
            return pltpu.async_copy(
                tab_hbm.at[idx_v.at[pl.ds(i * CHUNK, CHUNK)]], buf, gsem)

        def store(i, buf):
            return pltpu.async_copy(
                buf, out_hbm.at[pl.ds(base + i * CHUNK, CHUNK)], ssem)

        gather(0, rows_v.at[0])
        for i in range(NCHUNK):
            cur = rows_v.at[i % 2]
            nxt = rows_v.at[(i + 1) % 2]
            pltpu.make_async_copy(
                tab_hbm.at[idx_v.at[pl.ds(i * CHUNK, CHUNK)]], cur,
                gsem).wait()
            if i > 0:
                pltpu.make_async_copy(
                    rows_v.at[(i - 1) % 2],
                    out_hbm.at[pl.ds(base + (i - 1) * CHUNK, CHUNK)],
                    ssem).wait()
            if i + 1 < NCHUNK:
                gather(i + 1, nxt)
            store(i, cur)
        pltpu.make_async_copy(
            rows_v.at[(NCHUNK - 1) % 2],
            out_hbm.at[pl.ds(base + (NCHUNK - 1) * CHUNK, CHUNK)],
            ssem).wait()

    return _sc_gather


def _mlp_body(xn_ref, emb_ref, w1n_ref, w1e_ref, b1_ref, g1_ref, be1_ref,
              m1_ref, v1_ref, w2_ref, b2_ref, g2_ref, be2_ref, m2_ref,
              v2_ref, w3_ref, b3_ref, out_ref):
    h = jnp.dot(xn_ref[...], w1n_ref[...], preferred_element_type=jnp.float32)
    h = h + jnp.dot(emb_ref[...], w1e_ref[...],
                    preferred_element_type=jnp.float32)
    h = jnp.maximum(h + b1_ref[...], 0.0)
    h = (h - m1_ref[...]) / jnp.sqrt(v1_ref[...] + EPS) * g1_ref[...] \
        + be1_ref[...]
    h = jnp.dot(h, w2_ref[...], preferred_element_type=jnp.float32)
    h = jnp.maximum(h + b2_ref[...], 0.0)
    h = (h - m2_ref[...]) / jnp.sqrt(v2_ref[...] + EPS) * g2_ref[...] \
        + be2_ref[...]
    out_ref[...] = jnp.dot(h, w3_ref[...],
                           preferred_element_type=jnp.float32) + b3_ref[...]


BM = 1024


def _mlp(x_num, emb, w1n, w1e, b1, g1, be1, m1, v1, w2, b2, g2, be2, m2, v2,
         w3, b3):
    n_blocks = B // BM
    row_block = lambda i: (i, 0)
    full = lambda shape: pl.BlockSpec(shape, lambda i: (0, 0))
    return pl.pallas_call(
        _mlp_body,
        grid=(n_blocks,),
        in_specs=[
            pl.BlockSpec((BM, NNUM), row_block),
            pl.BlockSpec((BM, NF * ED), row_block),
            full((NNUM, H)),
            full((NF * ED, H)),
            full((1, H)), full((1, H)), full((1, H)), full((1, H)),
            full((1, H)),
            full((H, H // 2)),
            full((1, H // 2)), full((1, H // 2)), full((1, H // 2)),
            full((1, H // 2)), full((1, H // 2)),
            full((H // 2, 1)),
            full((1, 1)),
        ],
        out_specs=pl.BlockSpec((BM, 1), row_block),
        out_shape=jax.ShapeDtypeStruct((B, 1), jnp.float32),
    )(x_num, emb, w1n, w1e, b1, g1, be1, m1, v1, w2, b2, g2, be2, m2, v2,
      w3, b3)


def kernel(x_num, x_cat, tables, W1, b1, g1, be1, m1, v1, W2, b2, g2, be2,
           m2, v2, W3, b3):
    tabT = tables.transpose(0, 2, 1).reshape(NF * 2, 8, V)
    tail = jnp.pad(tabT[:, :, VTAIL:], ((0, 0), (0, 0), (0, 96)))
    lin = _sc_transpose_fn()(tabT, tail)
    flat_tab = lin.reshape(NF * VPAD, ED)
    idx = (x_cat
           + jnp.arange(NF, dtype=jnp.int32)[None, :] * VPAD).reshape(-1)
    emb = _sc_gather_fn()(flat_tab, idx).reshape(B, NF * ED)
    out = _mlp(x_num, emb,
               W1[:NNUM], W1[NNUM:],
               b1.reshape(1, H), g1.reshape(1, H), be1.reshape(1, H),
               m1.reshape(1, H), v1.reshape(1, H),
               W2,
               b2.reshape(1, H // 2), g2.reshape(1, H // 2),
               be2.reshape(1, H // 2), m2.reshape(1, H // 2),
               v2.reshape(1, H // 2),
               W3, b3.reshape(1, 1))
    return out


# speedup vs baseline: 1.1242x; 1.1242x over previous
import functools

import jax
import jax.numpy as jnp
from jax import lax
from jax.experimental import pallas as pl
from jax.experimental.pallas import tpu as pltpu
from jax.experimental.pallas import tpu_sc as plsc

B = 16384
NNUM = 13
NF = 26
V = 100000
ED = 16
H = 128
EPS = 1e-5

NC, NS = 2, 16
NW = NC * NS

RPF = 12504
VPAD = RPF * 8
NCH_T = 25
CH_T = 4096
CH_LAST = 1664
VTAIL = 99968
UNITS_T = NF * NCH_T
HALF_E = 2048

ROWS = B * NF
ROWS_PER_W = ROWS // NW
CHUNK = 1664
NCHUNK = ROWS_PER_W // CHUNK


QE = 1024
QW = QE * ED
FLUSH_B = QW * 4


@functools.cache
def _sc_transpose_fn():
    mesh = plsc.VectorSubcoreMesh(core_axis_name="c", subcore_axis_name="s")

    @functools.partial(
        pl.kernel,
        out_type=jax.ShapeDtypeStruct((NF, RPF, 128), jnp.float32),
        mesh=mesh,
        scratch_types=[
            pltpu.VMEM((2, 8, CH_T), jnp.float32),
            pltpu.VMEM((2, 8, 128), jnp.float32),
            pltpu.VMEM((QE // 8, 128), jnp.float32),
            pltpu.VMEM((QE // 8, 128), jnp.float32),
            pltpu.SemaphoreType.DMA,
            pltpu.SemaphoreType.DMA,
            pltpu.SemaphoreType.DMA,
        ],
        compiler_params=pltpu.CompilerParams(needs_layout_passes=False),
    )
    def _sc_transpose(tab_hbm, tail_hbm, out_hbm, slab, tslab, obufA, obufB,
                      bsem, fsem0, fsem1):
        wid = lax.axis_index("s") * NC + lax.axis_index("c")
        u_lo = (UNITS_T * wid) // NW
        u_hi = (UNITS_T * (wid + 1)) // NW
        iot = lax.iota(jnp.int32, 16)
        shalf = [slab.at[0], slab.at[1]]
        thalf = [tslab.at[0], tslab.at[1]]
        rowc = [jnp.full((16,), e % 8, jnp.int32) for e in range(16)]
        fsems = [fsem0, fsem1]
        obufs = [obufA, obufB]

        def blocks(ob, rows, col_base, li_base, n_ent):
            def bbody(i, carry):
                k = i * 16 + iot
                vcol = col_base + k
                lio = li_base + k
                rowv = lio >> 3
                colb = (lio & 7) << 4
                for e in range(16):
                    vals = plsc.load_gather(rows[e // 8], [rowc[e], vcol])
                    plsc.store_scatter(ob, [rowv, colb + e], vals)
                return carry

            lax.fori_loop(0, n_ent // 16, bbody, 0)

        def flush_wait(h):
            pltpu.make_async_copy(obufs[h], out_hbm.at[0, pl.ds(0, QE // 8)],
                                  fsems[h]).wait()

        def do_quarter(f, c0, h0, h, n_h):
            @pl.when(n_h >= 1)
            def _():
                flush_wait(h)
            blocks(obufs[h], shalf, h0, 0, QE)
            r0 = pl.multiple_of((c0 + h0) // 8, 8)
            pltpu.async_copy(obufs[h], out_hbm.at[f, pl.ds(r0, QE // 8)],
                             fsems[h])
            return n_h + 1

        def do_chunk(u, f, c0, is_tail, n0, n1):
            c0 = pl.multiple_of(c0, 1024)
            cw = CH_LAST if is_tail else CH_T
            pltpu.async_copy(tab_hbm.at[2 * f, :, pl.ds(c0, cw)],
                             slab.at[0, :, pl.ds(0, cw)], bsem)
            pltpu.async_copy(tab_hbm.at[2 * f + 1, :, pl.ds(c0, cw)],
                             slab.at[1, :, pl.ds(0, cw)], bsem)
            pltpu.make_async_copy(tab_hbm.at[2 * f, :, pl.ds(c0, cw)],
                                  slab.at[0, :, pl.ds(0, cw)], bsem).wait()
            pltpu.make_async_copy(tab_hbm.at[2 * f + 1, :, pl.ds(c0, cw)],
                                  slab.at[1, :, pl.ds(0, cw)], bsem).wait()
            if not is_tail:
                n0 = do_quarter(f, c0, 0 * QE, 0, n0)
                n1 = do_quarter(f, c0, 1 * QE, 1, n1)
                n0 = do_quarter(f, c0, 2 * QE, 0, n0)
                n1 = do_quarter(f, c0, 3 * QE, 1, n1)
            else:
                @pl.when(n0 >= 1)
                def _():
                    flush_wait(0)
                blocks(obufA, shalf, 0, 0, QE)
                r0 = pl.multiple_of(c0 // 8, 8)
                pltpu.sync_copy(obufA, out_hbm.at[f, pl.ds(r0, QE // 8)])
                blocks(obufA, shalf, QE, 0, CH_LAST - QE)
                pltpu.sync_copy(tail_hbm.at[2 * f], tslab.at[0])
                pltpu.sync_copy(tail_hbm.at[2 * f + 1], tslab.at[1])
                blocks(obufA, thalf, 0, CH_LAST - QE, 32)
                r2 = pl.multiple_of((c0 + QE) // 8, 8)
                pltpu.sync_copy(obufA.at[pl.ds(0, 88)],
                                out_hbm.at[f, pl.ds(r2, 88)])
                n0 = n0 * 0
            return n0, n1

        def body(u, carry):
            n0, n1 = carry
            f = u // NCH_T
            c = u % NCH_T
            is_tail = c == NCH_T - 1
            c0 = c * CH_T

            def tail_fn():
                return do_chunk(u, f, (NCH_T - 1) * CH_T, True, n0, n1)

            def reg_fn():
                return do_chunk(u, f, c0, False, n0, n1)

            return lax.cond(is_tail, tail_fn, reg_fn)

        n0, n1 = lax.fori_loop(u_lo, u_hi, body, (jnp.int32(0), jnp.int32(0)))

        @pl.when(n0 >= 1)
        def _():
            flush_wait(0)

        @pl.when(n1 >= 1)
        def _():
            flush_wait(1)

    return _sc_transpose


@functools.cache
def _sc_gather_fn():
    mesh = plsc.VectorSubcoreMesh(core_axis_name="c", subcore_axis_name="s")

    @functools.partial(
        pl.kernel,
        out_type=jax.ShapeDtypeStruct((ROWS, ED), jnp.float32),
        mesh=mesh,
        scratch_types=[
            pltpu.VMEM((ROWS_PER_W,), jnp.int32),
            pltpu.VMEM((2, CHUNK, ED), jnp.float32),
            pltpu.SemaphoreType.DMA,
            pltpu.SemaphoreType.DMA,
            pltpu.SemaphoreType.DMA,
        ],
        compiler_params=pltpu.CompilerParams(use_tc_tiling_on_sc=False),
    )
    def _sc_gather(tab_hbm, idx_hbm, out_hbm, idx_v, rows_v, gsem, ssem,
                   isem):
        wid = lax.axis_index("s") * NC + lax.axis_index("c")
        base = wid * ROWS_PER_W
        pltpu.async_copy(idx_hbm.at[pl.ds(base, ROWS_PER_W)], idx_v,
                         isem).wait()

        def gather(i, buf):
            return pltpu.async_copy(
                tab_hbm.at[idx_v.at[pl.ds(i * CHUNK, CHUNK)]], buf, gsem)

        def store(i, buf):
            return pltpu.async_copy(
                buf, out_hbm.at[pl.ds(base + i * CHUNK, CHUNK)], ssem)

        gather(0, rows_v.at[0])
        for i in range(NCHUNK):
            cur = rows_v.at[i % 2]
            nxt = rows_v.at[(i + 1) % 2]
            pltpu.make_async_copy(
                tab_hbm.at[idx_v.at[pl.ds(i * CHUNK, CHUNK)]], cur,
                gsem).wait()
            if i > 0:
                pltpu.make_async_copy(
                    rows_v.at[(i - 1) % 2],
                    out_hbm.at[pl.ds(base + (i - 1) * CHUNK, CHUNK)],
                    ssem).wait()
            if i + 1 < NCHUNK:
                gather(i + 1, nxt)
            store(i, cur)
        pltpu.make_async_copy(
            rows_v.at[(NCHUNK - 1) % 2],
            out_hbm.at[pl.ds(base + (NCHUNK - 1) * CHUNK, CHUNK)],
            ssem).wait()

    return _sc_gather


def _mlp_body(xn_ref, emb_ref, w1n_ref, w1e_ref, b1_ref, g1_ref, be1_ref,
              m1_ref, v1_ref, w2_ref, b2_ref, g2_ref, be2_ref, m2_ref,
              v2_ref, w3_ref, b3_ref, out_ref):
    h = jnp.dot(xn_ref[...], w1n_ref[...], preferred_element_type=jnp.float32)
    h = h + jnp.dot(emb_ref[...], w1e_ref[...],
                    preferred_element_type=jnp.float32)
    h = jnp.maximum(h + b1_ref[...], 0.0)
    h = (h - m1_ref[...]) / jnp.sqrt(v1_ref[...] + EPS) * g1_ref[...] \
        + be1_ref[...]
    h = jnp.dot(h, w2_ref[...], preferred_element_type=jnp.float32)
    h = jnp.maximum(h + b2_ref[...], 0.0)
    h = (h - m2_ref[...]) / jnp.sqrt(v2_ref[...] + EPS) * g2_ref[...] \
        + be2_ref[...]
    out_ref[...] = jnp.dot(h, w3_ref[...],
                           preferred_element_type=jnp.float32) + b3_ref[...]


BM = 1024


def _mlp(x_num, emb, w1n, w1e, b1, g1, be1, m1, v1, w2, b2, g2, be2, m2, v2,
         w3, b3):
    n_blocks = B // BM
    row_block = lambda i: (i, 0)
    full = lambda shape: pl.BlockSpec(shape, lambda i: (0, 0))
    return pl.pallas_call(
        _mlp_body,
        grid=(n_blocks,),
        in_specs=[
            pl.BlockSpec((BM, NNUM), row_block),
            pl.BlockSpec((BM, NF * ED), row_block),
            full((NNUM, H)),
            full((NF * ED, H)),
            full((1, H)), full((1, H)), full((1, H)), full((1, H)),
            full((1, H)),
            full((H, H // 2)),
            full((1, H // 2)), full((1, H // 2)), full((1, H // 2)),
            full((1, H // 2)), full((1, H // 2)),
            full((H // 2, 1)),
            full((1, 1)),
        ],
        out_specs=pl.BlockSpec((BM, 1), row_block),
        out_shape=jax.ShapeDtypeStruct((B, 1), jnp.float32),
    )(x_num, emb, w1n, w1e, b1, g1, be1, m1, v1, w2, b2, g2, be2, m2, v2,
      w3, b3)


def kernel(x_num, x_cat, tables, W1, b1, g1, be1, m1, v1, W2, b2, g2, be2,
           m2, v2, W3, b3):
    tabT = tables.transpose(0, 2, 1).reshape(NF * 2, 8, V)
    tail = jnp.pad(tabT[:, :, VTAIL:], ((0, 0), (0, 0), (0, 96)))
    lin = _sc_transpose_fn()(tabT, tail)
    flat_tab = lin.reshape(NF * VPAD, ED)
    idx = (x_cat
           + jnp.arange(NF, dtype=jnp.int32)[None, :] * VPAD).reshape(-1)
    emb = _sc_gather_fn()(flat_tab, idx).reshape(B, NF * ED)
    out = _mlp(x_num, emb,
               W1[:NNUM], W1[NNUM:],
               b1.reshape(1, H), g1.reshape(1, H), be1.reshape(1, H),
               m1.reshape(1, H), v1.reshape(1, H),
               W2,
               b2.reshape(1, H // 2), g2.reshape(1, H // 2),
               be2.reshape(1, H // 2), m2.reshape(1, H // 2),
               v2.reshape(1, H // 2),
               W3, b3.reshape(1, 1))
    return out


# speedup vs baseline: 1.1245x; 1.0003x over previous
import functools

import jax
import jax.numpy as jnp
from jax import lax
from jax.experimental import pallas as pl
from jax.experimental.pallas import tpu as pltpu
from jax.experimental.pallas import tpu_sc as plsc

B = 16384
NNUM = 13
NF = 26
V = 100000
ED = 16
H = 128
EPS = 1e-5

NC, NS = 2, 16
NW = NC * NS

RPF = 12504
VPAD = RPF * 8
NCH_T = 25
CH_T = 4096
CH_LAST = 1664
VTAIL = 99968
UNITS_T = NF * NCH_T
HALF_E = 2048

ROWS = B * NF
ROWS_PER_W = ROWS // NW
CHUNK = 1664
NCHUNK = ROWS_PER_W // CHUNK


QE = 1024
QW = QE * ED
FLUSH_B = QW * 4


@functools.cache
def _sc_transpose_fn():
    mesh = plsc.VectorSubcoreMesh(core_axis_name="c", subcore_axis_name="s")

    @functools.partial(
        pl.kernel,
        out_type=jax.ShapeDtypeStruct((NF * RPF * 128,), jnp.float32),
        mesh=mesh,
        scratch_types=[
            pltpu.VMEM((2, 8, CH_T), jnp.float32),
            pltpu.VMEM((2, 8, 128), jnp.float32),
            pltpu.VMEM((QW,), jnp.float32),
            pltpu.VMEM((QW,), jnp.float32),
            pltpu.SemaphoreType.DMA,
            pltpu.SemaphoreType.DMA,
            pltpu.SemaphoreType.DMA,
        ],
        compiler_params=pltpu.CompilerParams(needs_layout_passes=False),
    )
    def _sc_transpose(tab_hbm, tail_hbm, out_hbm, slab, tslab, obufA, obufB,
                      bsem, fsem0, fsem1):
        wid = lax.axis_index("s") * NC + lax.axis_index("c")
        u_lo = (UNITS_T * wid) // NW
        u_hi = (UNITS_T * (wid + 1)) // NW
        iot = lax.iota(jnp.int32, 16)
        shalf = [slab.at[0], slab.at[1]]
        thalf = [tslab.at[0], tslab.at[1]]
        rowc = [jnp.full((16,), e % 8, jnp.int32) for e in range(16)]
        fsems = [fsem0, fsem1]
        obufs = [obufA, obufB]

        def blocks(ob, rows, col_base, li_base, n_ent):
            def bbody(i, carry):
                k = i * 16 + iot
                vcol = col_base + k
                lio = li_base + k
                addr2 = ((lio >> 3) << 7) + ((lio & 7) << 4)
                for e in range(16):
                    vals = plsc.load_gather(rows[e // 8], [rowc[e], vcol])
                    plsc.store_scatter(ob, [addr2 + e], vals)
                return carry

            lax.fori_loop(0, n_ent // 16, bbody, 0)

        def flush_wait(h):
            pltpu.make_async_copy(obufs[h], out_hbm.at[pl.ds(0, QW)],
                                  fsems[h]).wait()

        def do_quarter(f, c0, h0, h, n_h):
            @pl.when(n_h >= 1)
            def _():
                flush_wait(h)
            blocks(obufs[h], shalf, h0, 0, QE)
            off = pl.multiple_of((f * RPF * 8 + c0 + h0) * ED, 1024)
            pltpu.async_copy(obufs[h], out_hbm.at[pl.ds(off, QW)],
                             fsems[h])
            return n_h + 1

        def do_chunk(u, f, c0, is_tail, n0, n1):
            c0 = pl.multiple_of(c0, 1024)
            cw = CH_LAST if is_tail else CH_T
            pltpu.async_copy(tab_hbm.at[2 * f, :, pl.ds(c0, cw)],
                             slab.at[0, :, pl.ds(0, cw)], bsem)
            pltpu.async_copy(tab_hbm.at[2 * f + 1, :, pl.ds(c0, cw)],
                             slab.at[1, :, pl.ds(0, cw)], bsem)
            pltpu.make_async_copy(tab_hbm.at[2 * f, :, pl.ds(c0, cw)],
                                  slab.at[0, :, pl.ds(0, cw)], bsem).wait()
            pltpu.make_async_copy(tab_hbm.at[2 * f + 1, :, pl.ds(c0, cw)],
                                  slab.at[1, :, pl.ds(0, cw)], bsem).wait()
            if not is_tail:
                n0 = do_quarter(f, c0, 0 * QE, 0, n0)
                n1 = do_quarter(f, c0, 1 * QE, 1, n1)
                n0 = do_quarter(f, c0, 2 * QE, 0, n0)
                n1 = do_quarter(f, c0, 3 * QE, 1, n1)
            else:
                @pl.when(n0 >= 1)
                def _():
                    flush_wait(0)
                blocks(obufA, shalf, 0, 0, QE)
                off = pl.multiple_of((f * RPF * 8 + c0) * ED, 1024)
                pltpu.sync_copy(obufA, out_hbm.at[pl.ds(off, QW)])
                blocks(obufA, shalf, QE, 0, CH_LAST - QE)
                pltpu.sync_copy(tail_hbm.at[2 * f], tslab.at[0])
                pltpu.sync_copy(tail_hbm.at[2 * f + 1], tslab.at[1])
                blocks(obufA, thalf, 0, CH_LAST - QE, 32)
                off2 = pl.multiple_of((f * RPF * 8 + c0 + QE) * ED,
                                      1024)
                pltpu.sync_copy(obufA.at[pl.ds(0, 88 * 128)],
                                out_hbm.at[pl.ds(off2, 88 * 128)])
                n0 = n0 * 0
            return n0, n1

        def body(u, carry):
            n0, n1 = carry
            f = u // NCH_T
            c = u % NCH_T
            is_tail = c == NCH_T - 1
            c0 = c * CH_T

            def tail_fn():
                return do_chunk(u, f, (NCH_T - 1) * CH_T, True, n0, n1)

            def reg_fn():
                return do_chunk(u, f, c0, False, n0, n1)

            return lax.cond(is_tail, tail_fn, reg_fn)

        n0, n1 = lax.fori_loop(u_lo, u_hi, body, (jnp.int32(0), jnp.int32(0)))

        @pl.when(n0 >= 1)
        def _():
            flush_wait(0)

        @pl.when(n1 >= 1)
        def _():
            flush_wait(1)

    return _sc_transpose


@functools.cache
def _sc_gather_fn():
    mesh = plsc.VectorSubcoreMesh(core_axis_name="c", subcore_axis_name="s")

    @functools.partial(
        pl.kernel,
        out_type=jax.ShapeDtypeStruct((ROWS, ED), jnp.float32),
        mesh=mesh,
        scratch_types=[
            pltpu.VMEM((ROWS_PER_W,), jnp.int32),
            pltpu.VMEM((2, CHUNK, ED), jnp.float32),
            pltpu.SemaphoreType.DMA,
            pltpu.SemaphoreType.DMA,
            pltpu.SemaphoreType.DMA,
        ],
        compiler_params=pltpu.CompilerParams(use_tc_tiling_on_sc=False),
    )
    def _sc_gather(tab_hbm, idx_hbm, out_hbm, idx_v, rows_v, gsem, ssem,
                   isem):
        wid = lax.axis_index("s") * NC + lax.axis_index("c")
        base = wid * ROWS_PER_W
        pltpu.async_copy(idx_hbm.at[pl.ds(base, ROWS_PER_W)], idx_v,
                         isem).wait()

        def gather(i, buf):
            return pltpu.async_copy(
                tab_hbm.at[idx_v.at[pl.ds(i * CHUNK, CHUNK)]], buf, gsem)

        def store(i, buf):
            return pltpu.async_copy(
                buf, out_hbm.at[pl.ds(base + i * CHUNK, CHUNK)], ssem)

        gather(0, rows_v.at[0])
        for i in range(NCHUNK):
            cur = rows_v.at[i % 2]
            nxt = rows_v.at[(i + 1) % 2]
            pltpu.make_async_copy(
                tab_hbm.at[idx_v.at[pl.ds(i * CHUNK, CHUNK)]], cur,
                gsem).wait()
            if i > 0:
                pltpu.make_async_copy(
                    rows_v.at[(i - 1) % 2],
                    out_hbm.at[pl.ds(base + (i - 1) * CHUNK, CHUNK)],
                    ssem).wait()
            if i + 1 < NCHUNK:
                gather(i + 1, nxt)
            store(i, cur)
        pltpu.make_async_copy(
            rows_v.at[(NCHUNK - 1) % 2],
            out_hbm.at[pl.ds(base + (NCHUNK - 1) * CHUNK, CHUNK)],
            ssem).wait()

    return _sc_gather


def _mlp_body(xn_ref, emb_ref, w1n_ref, w1e_ref, b1_ref, g1_ref, be1_ref,
              m1_ref, v1_ref, w2_ref, b2_ref, g2_ref, be2_ref, m2_ref,
              v2_ref, w3_ref, b3_ref, out_ref):
    h = jnp.dot(xn_ref[...], w1n_ref[...], preferred_element_type=jnp.float32)
    h = h + jnp.dot(emb_ref[...], w1e_ref[...],
                    preferred_element_type=jnp.float32)
    h = jnp.maximum(h + b1_ref[...], 0.0)
    h = (h - m1_ref[...]) / jnp.sqrt(v1_ref[...] + EPS) * g1_ref[...] \
        + be1_ref[...]
    h = jnp.dot(h, w2_ref[...], preferred_element_type=jnp.float32)
    h = jnp.maximum(h + b2_ref[...], 0.0)
    h = (h - m2_ref[...]) / jnp.sqrt(v2_ref[...] + EPS) * g2_ref[...] \
        + be2_ref[...]
    out_ref[...] = jnp.dot(h, w3_ref[...],
                           preferred_element_type=jnp.float32) + b3_ref[...]


BM = 1024


def _mlp(x_num, emb, w1n, w1e, b1, g1, be1, m1, v1, w2, b2, g2, be2, m2, v2,
         w3, b3):
    n_blocks = B // BM
    row_block = lambda i: (i, 0)
    full = lambda shape: pl.BlockSpec(shape, lambda i: (0, 0))
    return pl.pallas_call(
        _mlp_body,
        grid=(n_blocks,),
        in_specs=[
            pl.BlockSpec((BM, NNUM), row_block),
            pl.BlockSpec((BM, NF * ED), row_block),
            full((NNUM, H)),
            full((NF * ED, H)),
            full((1, H)), full((1, H)), full((1, H)), full((1, H)),
            full((1, H)),
            full((H, H // 2)),
            full((1, H // 2)), full((1, H // 2)), full((1, H // 2)),
            full((1, H // 2)), full((1, H // 2)),
            full((H // 2, 1)),
            full((1, 1)),
        ],
        out_specs=pl.BlockSpec((BM, 1), row_block),
        out_shape=jax.ShapeDtypeStruct((B, 1), jnp.float32),
    )(x_num, emb, w1n, w1e, b1, g1, be1, m1, v1, w2, b2, g2, be2, m2, v2,
      w3, b3)


def kernel(x_num, x_cat, tables, W1, b1, g1, be1, m1, v1, W2, b2, g2, be2,
           m2, v2, W3, b3):
    tabT = tables.transpose(0, 2, 1).reshape(NF * 2, 8, V)
    tail = jnp.pad(tabT[:, :, VTAIL:], ((0, 0), (0, 0), (0, 96)))
    lin = _sc_transpose_fn()(tabT, tail)
    flat_tab = lin.reshape(NF * VPAD, ED)
    idx = (x_cat
           + jnp.arange(NF, dtype=jnp.int32)[None, :] * VPAD).reshape(-1)
    emb = _sc_gather_fn()(flat_tab, idx).reshape(B, NF * ED)
    out = _mlp(x_num, emb,
               W1[:NNUM], W1[NNUM:],
               b1.reshape(1, H), g1.reshape(1, H), be1.reshape(1, H),
               m1.reshape(1, H), v1.reshape(1, H),
               W2,
               b2.reshape(1, H // 2), g2.reshape(1, H // 2),
               be2.reshape(1, H // 2), m2.reshape(1, H // 2),
               v2.reshape(1, H // 2),
               W3, b3.reshape(1, 1))
    return out


# speedup vs baseline: 1.1257x; 1.0010x over previous
import functools

import jax
import jax.numpy as jnp
from jax import lax
from jax.experimental import pallas as pl
from jax.experimental.pallas import tpu as pltpu
from jax.experimental.pallas import tpu_sc as plsc

B = 16384
NNUM = 13
NF = 26
V = 100000
ED = 16
H = 128
EPS = 1e-5

NC, NS = 2, 16
NW = NC * NS

RPF = 12504
VPAD = RPF * 8
NCH_T = 25
CH_T = 4096
CH_LAST = 1664
VTAIL = 99968
UNITS_T = NF * NCH_T

ROWS = B * NF
ROWS_PER_W = ROWS // NW
CHUNK = 1664
NCHUNK = ROWS_PER_W // CHUNK


QE = 1024
QW = QE * ED


@functools.cache
def _sc_transpose_fn():
    mesh = plsc.VectorSubcoreMesh(core_axis_name="c", subcore_axis_name="s")

    @functools.partial(
        pl.kernel,
        out_type=jax.ShapeDtypeStruct((NF * RPF * 128,), jnp.float32),
        mesh=mesh,
        scratch_types=[
            pltpu.VMEM((2, 8, CH_T), jnp.float32),
            pltpu.VMEM((2, 8, 128), jnp.float32),
            pltpu.VMEM((QW,), jnp.float32),
            pltpu.VMEM((QW,), jnp.float32),
            pltpu.SemaphoreType.DMA,
            pltpu.SemaphoreType.DMA,
            pltpu.SemaphoreType.DMA,
        ],
        compiler_params=pltpu.CompilerParams(needs_layout_passes=False),
    )
    def _sc_transpose(tab_hbm, tail_hbm, out_hbm, slab, tslab, obufA, obufB,
                      bsem, fsem0, fsem1):
        wid = lax.axis_index("s") * NC + lax.axis_index("c")
        u_lo = (UNITS_T * wid) // NW
        u_hi = (UNITS_T * (wid + 1)) // NW
        iot = lax.iota(jnp.int32, 16)
        shalf = [slab.at[0], slab.at[1]]
        thalf = [tslab.at[0], tslab.at[1]]
        rowc = [jnp.full((16,), e % 8, jnp.int32) for e in range(16)]
        fsems = [fsem0, fsem1]
        obufs = [obufA, obufB]

        def blocks(ob, rows, col_base, li_base, n_ent):
            def bbody(i, carry):
                k = i * 16 + iot
                vcol = col_base + k
                lio = li_base + k
                addr2 = ((lio >> 3) << 7) + ((lio & 7) << 4)
                for e in range(16):
                    vals = plsc.load_gather(rows[e // 8], [rowc[e], vcol])
                    plsc.store_scatter(ob, [addr2 + e], vals)
                return carry

            lax.fori_loop(0, n_ent // 16, bbody, 0)

        def flush_wait(h):
            pltpu.make_async_copy(obufs[h], out_hbm.at[pl.ds(0, QW)],
                                  fsems[h]).wait()

        def do_quarter(f, c0, h0, h, n_h):
            @pl.when(n_h >= 1)
            def _():
                flush_wait(h)
            blocks(obufs[h], shalf, h0, 0, QE)
            off = pl.multiple_of((f * RPF * 8 + c0 + h0) * ED, 1024)
            pltpu.async_copy(obufs[h], out_hbm.at[pl.ds(off, QW)],
                             fsems[h])
            return n_h + 1

        def do_chunk(u, f, c0, is_tail, n0, n1):
            c0 = pl.multiple_of(c0, 1024)
            cw = CH_LAST if is_tail else CH_T
            pltpu.async_copy(tab_hbm.at[2 * f, :, pl.ds(c0, cw)],
                             slab.at[0, :, pl.ds(0, cw)], bsem)
            pltpu.async_copy(tab_hbm.at[2 * f + 1, :, pl.ds(c0, cw)],
                             slab.at[1, :, pl.ds(0, cw)], bsem)
            pltpu.make_async_copy(tab_hbm.at[2 * f, :, pl.ds(c0, cw)],
                                  slab.at[0, :, pl.ds(0, cw)], bsem).wait()
            pltpu.make_async_copy(tab_hbm.at[2 * f + 1, :, pl.ds(c0, cw)],
                                  slab.at[1, :, pl.ds(0, cw)], bsem).wait()
            if not is_tail:
                n0 = do_quarter(f, c0, 0 * QE, 0, n0)
                n1 = do_quarter(f, c0, 1 * QE, 1, n1)
                n0 = do_quarter(f, c0, 2 * QE, 0, n0)
                n1 = do_quarter(f, c0, 3 * QE, 1, n1)
            else:
                @pl.when(n0 >= 1)
                def _():
                    flush_wait(0)
                blocks(obufA, shalf, 0, 0, QE)
                off = pl.multiple_of((f * RPF * 8 + c0) * ED, 1024)
                pltpu.sync_copy(obufA, out_hbm.at[pl.ds(off, QW)])
                blocks(obufA, shalf, QE, 0, CH_LAST - QE)
                pltpu.sync_copy(tail_hbm.at[2 * f], tslab.at[0])
                pltpu.sync_copy(tail_hbm.at[2 * f + 1], tslab.at[1])
                blocks(obufA, thalf, 0, CH_LAST - QE, 32)
                off2 = pl.multiple_of((f * RPF * 8 + c0 + QE) * ED,
                                      1024)
                pltpu.sync_copy(obufA.at[pl.ds(0, 88 * 128)],
                                out_hbm.at[pl.ds(off2, 88 * 128)])
                n0 = n0 * 0
            return n0, n1

        def body(u, carry):
            n0, n1 = carry
            f = u // NCH_T
            c = u % NCH_T
            is_tail = c == NCH_T - 1
            c0 = c * CH_T

            def tail_fn():
                return do_chunk(u, f, (NCH_T - 1) * CH_T, True, n0, n1)

            def reg_fn():
                return do_chunk(u, f, c0, False, n0, n1)

            return lax.cond(is_tail, tail_fn, reg_fn)

        n0, n1 = lax.fori_loop(u_lo, u_hi, body, (jnp.int32(0), jnp.int32(0)))

        @pl.when(n0 >= 1)
        def _():
            flush_wait(0)

        @pl.when(n1 >= 1)
        def _():
            flush_wait(1)

    return _sc_transpose


@functools.cache
def _sc_gather_fn():
    mesh = plsc.VectorSubcoreMesh(core_axis_name="c", subcore_axis_name="s")

    @functools.partial(
        pl.kernel,
        out_type=jax.ShapeDtypeStruct((ROWS, ED), jnp.float32),
        mesh=mesh,
        scratch_types=[
            pltpu.VMEM((ROWS_PER_W,), jnp.int32),
            pltpu.VMEM((2, CHUNK, ED), jnp.float32),
            pltpu.SemaphoreType.DMA,
            pltpu.SemaphoreType.DMA,
            pltpu.SemaphoreType.DMA,
        ],
        compiler_params=pltpu.CompilerParams(use_tc_tiling_on_sc=False),
    )
    def _sc_gather(tab_hbm, idx_hbm, out_hbm, idx_v, rows_v, gsem, ssem,
                   isem):
        wid = lax.axis_index("s") * NC + lax.axis_index("c")
        base = wid * ROWS_PER_W
        pltpu.async_copy(idx_hbm.at[pl.ds(base, ROWS_PER_W)], idx_v,
                         isem).wait()

        def gather(i, buf):
            return pltpu.async_copy(
                tab_hbm.at[idx_v.at[pl.ds(i * CHUNK, CHUNK)]], buf, gsem)

        def store(i, buf):
            return pltpu.async_copy(
                buf, out_hbm.at[pl.ds(base + i * CHUNK, CHUNK)], ssem)

        gather(0, rows_v.at[0])
        for i in range(NCHUNK):
            cur = rows_v.at[i % 2]
            nxt = rows_v.at[(i + 1) % 2]
            pltpu.make_async_copy(
                tab_hbm.at[idx_v.at[pl.ds(i * CHUNK, CHUNK)]], cur,
                gsem).wait()
            if i > 0:
                pltpu.make_async_copy(
                    rows_v.at[(i - 1) % 2],
                    out_hbm.at[pl.ds(base + (i - 1) * CHUNK, CHUNK)],
                    ssem).wait()
            if i + 1 < NCHUNK:
                gather(i + 1, nxt)
            store(i, cur)
        pltpu.make_async_copy(
            rows_v.at[(NCHUNK - 1) % 2],
            out_hbm.at[pl.ds(base + (NCHUNK - 1) * CHUNK, CHUNK)],
            ssem).wait()

    return _sc_gather


def _mlp_body(xn_ref, emb_ref, w1n_ref, w1e_ref, b1_ref, g1_ref, be1_ref,
              m1_ref, v1_ref, w2_ref, b2_ref, g2_ref, be2_ref, m2_ref,
              v2_ref, w3_ref, b3_ref, out_ref):
    h = jnp.dot(xn_ref[...], w1n_ref[...], preferred_element_type=jnp.float32)
    h = h + jnp.dot(emb_ref[...], w1e_ref[...],
                    preferred_element_type=jnp.float32)
    h = jnp.maximum(h + b1_ref[...], 0.0)
    h = (h - m1_ref[...]) / jnp.sqrt(v1_ref[...] + EPS) * g1_ref[...] \
        + be1_ref[...]
    h = jnp.dot(h, w2_ref[...], preferred_element_type=jnp.float32)
    h = jnp.maximum(h + b2_ref[...], 0.0)
    h = (h - m2_ref[...]) / jnp.sqrt(v2_ref[...] + EPS) * g2_ref[...] \
        + be2_ref[...]
    out_ref[...] = jnp.dot(h, w3_ref[...],
                           preferred_element_type=jnp.float32) + b3_ref[...]


BM = 1024


def _mlp(x_num, emb, w1n, w1e, b1, g1, be1, m1, v1, w2, b2, g2, be2, m2, v2,
         w3, b3):
    n_blocks = B // BM
    row_block = lambda i: (i, 0)
    full = lambda shape: pl.BlockSpec(shape, lambda i: (0, 0))
    return pl.pallas_call(
        _mlp_body,
        grid=(n_blocks,),
        in_specs=[
            pl.BlockSpec((BM, NNUM), row_block),
            pl.BlockSpec((BM, NF * ED), row_block),
            full((NNUM, H)),
            full((NF * ED, H)),
            full((1, H)), full((1, H)), full((1, H)), full((1, H)),
            full((1, H)),
            full((H, H // 2)),
            full((1, H // 2)), full((1, H // 2)), full((1, H // 2)),
            full((1, H // 2)), full((1, H // 2)),
            full((H // 2, 1)),
            full((1, 1)),
        ],
        out_specs=pl.BlockSpec((BM, 1), row_block),
        out_shape=jax.ShapeDtypeStruct((B, 1), jnp.float32),
    )(x_num, emb, w1n, w1e, b1, g1, be1, m1, v1, w2, b2, g2, be2, m2, v2,
      w3, b3)


def kernel(x_num, x_cat, tables, W1, b1, g1, be1, m1, v1, W2, b2, g2, be2,
           m2, v2, W3, b3):
    tabT = tables.transpose(0, 2, 1).reshape(NF * 2, 8, V)
    tail = jnp.pad(tabT[:, :, VTAIL:], ((0, 0), (0, 0), (0, 96)))
    lin = _sc_transpose_fn()(tabT, tail)
    flat_tab = lin.reshape(NF * VPAD, ED)
    idx = (x_cat
           + jnp.arange(NF, dtype=jnp.int32)[None, :] * VPAD).reshape(-1)
    emb = _sc_gather_fn()(flat_tab, idx).reshape(B, NF * ED)
    out = _mlp(x_num, emb,
               W1[:NNUM], W1[NNUM:],
               b1.reshape(1, H), g1.reshape(1, H), be1.reshape(1, H),
               m1.reshape(1, H), v1.reshape(1, H),
               W2,
               b2.reshape(1, H // 2), g2.reshape(1, H // 2),
               be2.reshape(1, H // 2), m2.reshape(1, H // 2),
               v2.reshape(1, H // 2),
               W3, b3.reshape(1, 1))
    return out
